# Initial kernel scaffold; baseline (speedup 1.0000x reference)
import jax
import jax.numpy as jnp
from jax.experimental import pallas as pl


def kernel(weight_q, scale, grid):
    raise NotImplementedError("write your pallas kernel here")


# speedup vs baseline: 34.8077x; 34.8077x over previous
import functools

import jax
import jax.numpy as jnp
from jax import lax
from jax.experimental import pallas as pl
from jax.experimental.pallas import tpu as pltpu
from jax.experimental.pallas import tpu_sc as plsc

OUT_F = 4096
IN_F = 4096
CODESZ = 8
GRID_K = 65536

NC = 2
NS = 16
NW = NC * NS

B = OUT_F * (IN_F // CODESZ)
B_PER_W = B // NW
CHUNK = 8192
N_CHUNKS = B_PER_W // CHUNK


def _scale_body(s_ref, g_ref, o_ref):
    o_ref[...] = g_ref[...] * s_ref[0]


def _scale_grid(scale, grid2d):
    return pl.pallas_call(
        _scale_body,
        out_shape=jax.ShapeDtypeStruct(grid2d.shape, jnp.float32),
        in_specs=[
            pl.BlockSpec(memory_space=pltpu.SMEM),
            pl.BlockSpec(memory_space=pltpu.VMEM),
        ],
        out_specs=pl.BlockSpec(memory_space=pltpu.VMEM),
    )(scale, grid2d)


def _gather_body(table_hbm, idx_hbm, out_hbm, idx_v, rows_v, sem):
    wid = lax.axis_index("s") * NC + lax.axis_index("c")
    base = wid * B_PER_W

    def chunk_step(g, carry):
        off = base + g * CHUNK
        pltpu.sync_copy(idx_hbm.at[pl.ds(off, CHUNK)], idx_v)
        pltpu.async_copy(table_hbm.at[idx_v], rows_v, sem).wait()
        pltpu.sync_copy(rows_v, out_hbm.at[pl.ds(off, CHUNK)])
        return carry

    lax.fori_loop(0, N_CHUNKS, chunk_step, 0)


_gather_call = functools.partial(
    pl.kernel,
    out_type=jax.ShapeDtypeStruct((B, CODESZ), jnp.float32),
    mesh=plsc.VectorSubcoreMesh(core_axis_name="c", subcore_axis_name="s"),
    scratch_types=[
        pltpu.VMEM((CHUNK,), jnp.int32),
        pltpu.VMEM((CHUNK, CODESZ), jnp.float32),
        pltpu.SemaphoreType.DMA,
    ],
    compiler_params=pltpu.CompilerParams(use_tc_tiling_on_sc=False),
)(_gather_body)


def kernel(weight_q, scale, grid):
    idx = weight_q.astype(jnp.int32).reshape(-1)
    scaled = _scale_grid(scale, grid.reshape(GRID_K // 16, CODESZ * 16))
    out = _gather_call(scaled.reshape(GRID_K, CODESZ), idx)
    return out.reshape(OUT_F, IN_F)


# speedup vs baseline: 36.0209x; 1.0349x over previous
import functools

import jax
import jax.numpy as jnp
from jax import lax
from jax.experimental import pallas as pl
from jax.experimental.pallas import tpu as pltpu
from jax.experimental.pallas import tpu_sc as plsc

OUT_F = 4096
IN_F = 4096
CODESZ = 8
GRID_K = 65536

NC = 2
NS = 16
NW = NC * NS

B = OUT_F * (IN_F // CODESZ)
B_PER_W = B // NW
CHUNK = 4096
N_CHUNKS = B_PER_W // CHUNK
NBUF = 2


def _scale_body(s_ref, g_ref, o_ref):
    o_ref[...] = g_ref[...] * s_ref[0]


def _scale_grid(scale, grid2d):
    return pl.pallas_call(
        _scale_body,
        out_shape=jax.ShapeDtypeStruct(grid2d.shape, jnp.float32),
        in_specs=[
            pl.BlockSpec(memory_space=pltpu.SMEM),
            pl.BlockSpec(memory_space=pltpu.VMEM),
        ],
        out_specs=pl.BlockSpec(memory_space=pltpu.VMEM),
    )(scale, grid2d)


def _gather_body(table_hbm, idx_hbm, out_hbm,
                 idx0, idx1, rows0, rows1,
                 si0, si1, sg0, sg1, sw0, sw1):
    wid = lax.axis_index("s") * NC + lax.axis_index("c")
    base = wid * B_PER_W
    idx_v = (idx0, idx1)
    rows_v = (rows0, rows1)
    sem_i = (si0, si1)
    sem_g = (sg0, sg1)
    sem_w = (sw0, sw1)

    def start_idx(g, b):
        off = base + g * CHUNK
        return pltpu.async_copy(idx_hbm.at[pl.ds(off, CHUNK)], idx_v[b],
                                sem_i[b])

    def start_gather(b):
        return pltpu.async_copy(table_hbm.at[idx_v[b]], rows_v[b], sem_g[b])

    def start_write(g, b):
        off = base + g * CHUNK
        return pltpu.async_copy(rows_v[b], out_hbm.at[pl.ds(off, CHUNK)],
                                sem_w[b])

    pend_i = [start_idx(0, 0), start_idx(1, 1)]
    pend_g = [None, None]
    pend_w = [None, None]
    for g in range(N_CHUNKS):
        b = g % NBUF
        pend_i[b].wait()
        if pend_w[b] is not None:
            pend_w[b].wait()
        pend_g[b] = start_gather(b)
        pend_g[b].wait()
        pend_w[b] = start_write(g, b)
        if g + NBUF < N_CHUNKS:
            pend_i[b] = start_idx(g + NBUF, b)
    for b in range(NBUF):
        if pend_w[b] is not None:
            pend_w[b].wait()


_gather_call = functools.partial(
    pl.kernel,
    out_type=jax.ShapeDtypeStruct((B, CODESZ), jnp.float32),
    mesh=plsc.VectorSubcoreMesh(core_axis_name="c", subcore_axis_name="s"),
    scratch_types=[
        pltpu.VMEM((CHUNK,), jnp.int32),
        pltpu.VMEM((CHUNK,), jnp.int32),
        pltpu.VMEM((CHUNK, CODESZ), jnp.float32),
        pltpu.VMEM((CHUNK, CODESZ), jnp.float32),
        pltpu.SemaphoreType.DMA,
        pltpu.SemaphoreType.DMA,
        pltpu.SemaphoreType.DMA,
        pltpu.SemaphoreType.DMA,
        pltpu.SemaphoreType.DMA,
        pltpu.SemaphoreType.DMA,
    ],
    compiler_params=pltpu.CompilerParams(use_tc_tiling_on_sc=False),
)(_gather_body)


def kernel(weight_q, scale, grid):
    idx = weight_q.astype(jnp.int32).reshape(-1)
    scaled = _scale_grid(scale, grid.reshape(GRID_K // 16, CODESZ * 16))
    out = _gather_call(scaled.reshape(GRID_K, CODESZ), idx)
    return out.reshape(OUT_F, IN_F)
